# baseline (speedup 1.0000x reference)
import functools

import jax
import jax.numpy as jnp
from jax import lax
from jax.experimental import pallas as pl
from jax.experimental.pallas import tpu as pltpu
from jax.experimental.pallas import tpu_sc as plsc

_INTERPRET = False

B = 16
N = 32768
D = 64
RB = 8192
NV = N // 16


def _sumsq_body(w_ref, ss_ref):
    x = w_ref[...]
    ss_ref[...] = jnp.sum(x * x, axis=1, keepdims=True)


def _mask_body(w_ref, ss_ref, thr_ref, out_ref):
    i = pl.program_id(0)
    t = thr_ref[i, 0]
    m = (ss_ref[...] >= t).astype(jnp.float32)
    out_ref[...] = w_ref[...] * m


def _splat(x):
    return jnp.broadcast_to(x, (16,))


def _scan_hist(hist_ref, nbuckets, t_minus_k):
    lanes = lax.iota(jnp.int32, 16)

    def body(i, carry):
        carry_cnt, best_j, best_pre, best_cnt = carry
        v = hist_ref[pl.ds(i * 16, 16)]
        c = plsc.cumsum(v)
        pre_excl = carry_cnt + c - v
        cond = pre_excl <= t_minus_k
        p = _splat(plsc.all_reduce_population_count(cond))
        has = p > 0
        lane = p - 1
        sel = lanes == lane
        v_at = _splat(jnp.sum(jnp.where(sel, v, 0)))
        pre_at = _splat(jnp.sum(jnp.where(sel, pre_excl, 0)))
        best_j = jnp.where(has, i * 16 + lane, best_j)
        best_pre = jnp.where(has, pre_at, best_pre)
        best_cnt = jnp.where(has, v_at, best_cnt)
        carry_cnt = carry_cnt + _splat(jnp.sum(jnp.where(lanes == 15, c, 0)))
        return carry_cnt, best_j, best_pre, best_cnt

    z = jnp.zeros((16,), jnp.int32)
    _, j, pre, cnt = lax.fori_loop(0, nbuckets // 16, body, (z, z, z, z))
    return j, pre, cnt


def _zero_hist(hist_ref, nbuckets):
    def body(i, carry):
        hist_ref[pl.ds(i * 16, 16)] = jnp.zeros((16,), jnp.int32)
        return carry
    lax.fori_loop(0, nbuckets // 16, body, 0)


def _hist_pass(ss_ref, hist_ref, shift, nmaskbits, prefix):
    ones = jnp.ones((16,), jnp.int32)

    def body(i, carry):
        v = ss_ref[pl.ds(i * 16, 16)]
        bits = plsc.bitcast(v, jnp.int32)
        if nmaskbits:
            hi = lax.shift_right_logical(bits, shift + nmaskbits)
            m = hi == prefix
            bucket = lax.shift_right_logical(bits, shift) & ((1 << nmaskbits) - 1)
            plsc.addupdate_scatter(hist_ref, [bucket], ones, mask=m)
        else:
            bucket = lax.shift_right_logical(bits, shift)
            plsc.addupdate_scatter(hist_ref, [bucket], ones)
        return carry

    lax.fori_loop(0, NV, body, 0)


def _sc_body(ss_hbm, k_hbm, out_hbm, ss_v, k_v, hist_v, thr_v):
    c = lax.axis_index("c")
    s = lax.axis_index("s")

    @pl.when(c == 0)
    def _():
        pltpu.sync_copy(ss_hbm.at[s], ss_v)
        pltpu.sync_copy(k_hbm.at[s], k_v)
        k = k_v[...]

        _zero_hist(hist_v, 2048)
        _hist_pass(ss_v, hist_v, 20, 0, 0)
        b1, pre1, cnt1 = _scan_hist(hist_v, 2048, N - k)
        k2 = k - (N - (pre1 + cnt1))
        _zero_hist(hist_v, 1024)
        _hist_pass(ss_v, hist_v, 10, 10, b1)
        b2, pre2, cnt2 = _scan_hist(hist_v, 1024, cnt1 - k2)
        k3 = k2 - (cnt1 - (pre2 + cnt2))
        _zero_hist(hist_v, 1024)
        _hist_pass(ss_v, hist_v, 0, 10, (b1 << 10) | b2)
        b3, _, _ = _scan_hist(hist_v, 1024, cnt2 - k3)

        thr_bits = (b1 << 20) | (b2 << 10) | b3
        thr_v[...] = plsc.bitcast(thr_bits, jnp.float32)
        pltpu.sync_copy(thr_v, out_hbm.at[s])


def _sc_select(ss, kvec):
    mesh = plsc.VectorSubcoreMesh(core_axis_name="c", subcore_axis_name="s")
    f = functools.partial(
        pl.kernel,
        mesh=mesh,
        out_type=jax.ShapeDtypeStruct((B, 16), jnp.float32),
        compiler_params=pltpu.CompilerParams(needs_layout_passes=False),
        scratch_types=[
            pltpu.VMEM((N,), jnp.float32),
            pltpu.VMEM((16,), jnp.int32),
            pltpu.VMEM((2048,), jnp.int32),
            pltpu.VMEM((16,), jnp.float32),
        ],
    )(_sc_body)
    return f(ss, kvec)


@jax.jit
def kernel(weight_params, logits, W, b):
    nblk = N // RB
    wt = jnp.swapaxes(weight_params, 1, 2)

    ss = pl.pallas_call(
        _sumsq_body,
        grid=(B, nblk),
        in_specs=[pl.BlockSpec((1, D, RB), lambda i, j: (i, 0, j))],
        out_specs=pl.BlockSpec((1, 1, RB), lambda i, j: (i, 0, j)),
        out_shape=jax.ShapeDtypeStruct((B, 1, N), jnp.float32),
        compiler_params=pltpu.CompilerParams(
            dimension_semantics=("parallel", "parallel")),
        interpret=_INTERPRET,
    )(wt)

    score = jax.nn.sigmoid(logits @ W + b)
    k = jnp.clip((score * N).astype(jnp.int32), 1, None)
    kvec = jnp.broadcast_to(k, (B, 16)).astype(jnp.int32)

    thresholds = _sc_select(ss.reshape(B, N), kvec)

    out_t = pl.pallas_call(
        _mask_body,
        grid=(B, nblk),
        in_specs=[
            pl.BlockSpec((1, D, RB), lambda i, j: (i, 0, j)),
            pl.BlockSpec((1, 1, RB), lambda i, j: (i, 0, j)),
            pl.BlockSpec(memory_space=pltpu.SMEM),
        ],
        out_specs=pl.BlockSpec((1, D, RB), lambda i, j: (i, 0, j)),
        out_shape=jax.ShapeDtypeStruct((B, D, N), jnp.float32),
        compiler_params=pltpu.CompilerParams(
            dimension_semantics=("parallel", "parallel")),
        interpret=_INTERPRET,
    )(wt, ss, thresholds)

    return jnp.swapaxes(out_t, 1, 2)


# speedup vs baseline: 1.0273x; 1.0273x over previous
import functools

import jax
import jax.numpy as jnp
from jax import lax
from jax.experimental import pallas as pl
from jax.experimental.pallas import tpu as pltpu
from jax.experimental.pallas import tpu_sc as plsc

_INTERPRET = False

B = 16
N = 32768
D = 64
RB = 8192
NV = N // 16


def _sumsq_body(w_ref, ss_ref):
    x = w_ref[...]
    ss_ref[...] = jnp.sum(x * x, axis=1, keepdims=True)


def _mask_body(w_ref, ss_ref, thr_ref, out_ref):
    i = pl.program_id(0)
    t = thr_ref[i, 0]
    m = (ss_ref[...] >= t).astype(jnp.float32)
    out_ref[...] = w_ref[...] * m


def _splat(x):
    return jnp.broadcast_to(x, (16,))


def _scan_hist(hist_ref, nbuckets, t_minus_k):
    lanes = lax.iota(jnp.int32, 16)

    def body(i, carry):
        carry_cnt, best_j, best_pre, best_cnt = carry
        v = hist_ref[pl.ds(i * 16, 16)]
        c = plsc.cumsum(v)
        pre_excl = carry_cnt + c - v
        cond = pre_excl <= t_minus_k
        p = _splat(plsc.all_reduce_population_count(cond))
        has = p > 0
        lane = p - 1
        sel = lanes == lane
        v_at = _splat(jnp.sum(jnp.where(sel, v, 0)))
        pre_at = _splat(jnp.sum(jnp.where(sel, pre_excl, 0)))
        best_j = jnp.where(has, i * 16 + lane, best_j)
        best_pre = jnp.where(has, pre_at, best_pre)
        best_cnt = jnp.where(has, v_at, best_cnt)
        carry_cnt = carry_cnt + _splat(jnp.sum(jnp.where(lanes == 15, c, 0)))
        return carry_cnt, best_j, best_pre, best_cnt

    z = jnp.zeros((16,), jnp.int32)
    _, j, pre, cnt = lax.fori_loop(0, nbuckets // 16, body, (z, z, z, z),
                                   unroll=4)
    return j, pre, cnt


def _zero_hist(hist_ref, nbuckets):
    def body(i, carry):
        hist_ref[pl.ds(i * 16, 16)] = jnp.zeros((16,), jnp.int32)
        return carry
    lax.fori_loop(0, nbuckets // 16, body, 0, unroll=8)


def _hist_pass(ss_ref, hist_ref, shift, nmaskbits, prefix):
    ones = jnp.ones((16,), jnp.int32)

    def body(i, carry):
        v = ss_ref[pl.ds(i * 16, 16)]
        bits = plsc.bitcast(v, jnp.int32)
        if nmaskbits:
            hi = lax.shift_right_logical(bits, shift + nmaskbits)
            m = hi == prefix
            bucket = lax.shift_right_logical(bits, shift) & ((1 << nmaskbits) - 1)
            plsc.addupdate_scatter(hist_ref, [bucket], ones, mask=m)
        else:
            bucket = lax.shift_right_logical(bits, shift)
            plsc.addupdate_scatter(hist_ref, [bucket], ones)
        return carry

    lax.fori_loop(0, NV, body, 0, unroll=8)


def _sc_body(ss_hbm, k_hbm, out_hbm, ss_v, k_v, hist_v, thr_v):
    c = lax.axis_index("c")
    s = lax.axis_index("s")

    @pl.when(c == 0)
    def _():
        pltpu.sync_copy(ss_hbm.at[s], ss_v)
        pltpu.sync_copy(k_hbm.at[s], k_v)
        k = k_v[...]

        _zero_hist(hist_v, 2048)
        _hist_pass(ss_v, hist_v, 20, 0, 0)
        b1, pre1, cnt1 = _scan_hist(hist_v, 2048, N - k)
        k2 = k - (N - (pre1 + cnt1))
        _zero_hist(hist_v, 1024)
        _hist_pass(ss_v, hist_v, 10, 10, b1)
        b2, pre2, cnt2 = _scan_hist(hist_v, 1024, cnt1 - k2)
        k3 = k2 - (cnt1 - (pre2 + cnt2))
        _zero_hist(hist_v, 1024)
        _hist_pass(ss_v, hist_v, 0, 10, (b1 << 10) | b2)
        b3, _, _ = _scan_hist(hist_v, 1024, cnt2 - k3)

        thr_bits = (b1 << 20) | (b2 << 10) | b3
        thr_v[...] = plsc.bitcast(thr_bits, jnp.float32)
        pltpu.sync_copy(thr_v, out_hbm.at[s])


def _sc_select(ss, kvec):
    mesh = plsc.VectorSubcoreMesh(core_axis_name="c", subcore_axis_name="s")
    f = functools.partial(
        pl.kernel,
        mesh=mesh,
        out_type=jax.ShapeDtypeStruct((B, 16), jnp.float32),
        compiler_params=pltpu.CompilerParams(needs_layout_passes=False),
        scratch_types=[
            pltpu.VMEM((N,), jnp.float32),
            pltpu.VMEM((16,), jnp.int32),
            pltpu.VMEM((2048,), jnp.int32),
            pltpu.VMEM((16,), jnp.float32),
        ],
    )(_sc_body)
    return f(ss, kvec)


@jax.jit
def kernel(weight_params, logits, W, b):
    nblk = N // RB
    wt = jnp.swapaxes(weight_params, 1, 2)

    ss = pl.pallas_call(
        _sumsq_body,
        grid=(B, nblk),
        in_specs=[pl.BlockSpec((1, D, RB), lambda i, j: (i, 0, j))],
        out_specs=pl.BlockSpec((1, 1, RB), lambda i, j: (i, 0, j)),
        out_shape=jax.ShapeDtypeStruct((B, 1, N), jnp.float32),
        compiler_params=pltpu.CompilerParams(
            dimension_semantics=("parallel", "parallel")),
        interpret=_INTERPRET,
    )(wt)

    score = jax.nn.sigmoid(logits @ W + b)
    k = jnp.clip((score * N).astype(jnp.int32), 1, None)
    kvec = jnp.broadcast_to(k, (B, 16)).astype(jnp.int32)

    thresholds = _sc_select(ss.reshape(B, N), kvec)

    out_t = pl.pallas_call(
        _mask_body,
        grid=(B, nblk),
        in_specs=[
            pl.BlockSpec((1, D, RB), lambda i, j: (i, 0, j)),
            pl.BlockSpec((1, 1, RB), lambda i, j: (i, 0, j)),
            pl.BlockSpec(memory_space=pltpu.SMEM),
        ],
        out_specs=pl.BlockSpec((1, D, RB), lambda i, j: (i, 0, j)),
        out_shape=jax.ShapeDtypeStruct((B, D, N), jnp.float32),
        compiler_params=pltpu.CompilerParams(
            dimension_semantics=("parallel", "parallel")),
        interpret=_INTERPRET,
    )(wt, ss, thresholds)

    return jnp.swapaxes(out_t, 1, 2)
